# baseline (speedup 1.0000x reference)
import functools

import jax
import jax.numpy as jnp
from jax import lax
from jax.experimental import pallas as pl
from jax.experimental.pallas import tpu as pltpu
from jax.experimental.pallas import tpu_sc as plsc

NUM_STEPS = 8
EMBED = 16
HID = 96
N_NODES = 20736
N_EDGES = 414720

NC = 2
NS = 16
NW = NC * NS
CH = 128

BE = 1024
EP = 417792
NB_E = EP // BE
NB_REAL = N_EDGES // BE
BN = 256
NB_N = N_NODES // BN
CW = 32

def _mesh():
    return plsc.VectorSubcoreMesh(core_axis_name="c", subcore_axis_name="s",
                                  num_cores=NC, num_subcores=NS)


GUF = 4


def _sc_gather(ptab, qtab, idx2, n_rows):
    per_w = n_rows // NW
    n_ch = per_w // CH
    n_it = n_ch // GUF
    width = ptab.shape[1]
    dt = ptab.dtype

    @functools.partial(
        pl.kernel,
        out_type=jax.ShapeDtypeStruct((n_rows, width), dt),
        mesh=_mesh(),
        compiler_params=pltpu.CompilerParams(use_tc_tiling_on_sc=False),
        scratch_types=[
            pltpu.VMEM((n_ch, CH), jnp.int32),
            pltpu.VMEM((GUF, CH, width), dt),
        ] + [pltpu.SemaphoreType.DMA] * (2 * GUF),
    )
    def k(ptab_hbm, qtab_hbm, idx_hbm, out_hbm, idx_v, rows, *sems):
        gsem = sems[:GUF]
        wsem = sems[GUF:]
        wid = lax.axis_index("s") * NC + lax.axis_index("c")
        base = wid * per_w
        pltpu.sync_copy(idx_hbm.at[pl.ds(wid * n_ch, n_ch)], idx_v)

        def run(table_hbm):
            def body(g, carry):
                for b in range(GUF):
                    j = g * GUF + b

                    @pl.when(g >= 1)
                    def _():
                        off = base + (j - GUF) * CH
                        pltpu.make_async_copy(
                            rows.at[b], out_hbm.at[pl.ds(off, CH)],
                            wsem[b]).wait()

                    pltpu.async_copy(table_hbm.at[idx_v.at[j]], rows.at[b],
                                     gsem[b])
                for b in range(GUF):
                    j = g * GUF + b
                    pltpu.make_async_copy(
                        table_hbm.at[idx_v.at[j]], rows.at[b], gsem[b]).wait()
                    pltpu.async_copy(
                        rows.at[b], out_hbm.at[pl.ds(base + j * CH, CH)],
                        wsem[b])
                return carry

            lax.fori_loop(0, n_it, body, 0, unroll=False)
            for b in range(GUF):
                j = (n_it - 1) * GUF + b
                pltpu.make_async_copy(
                    rows.at[b], out_hbm.at[pl.ds(base + j * CH, CH)],
                    wsem[b]).wait()

        @pl.when(wid < NW // 2)
        def _():
            run(ptab_hbm)

        @pl.when(wid >= NW // 2)
        def _():
            run(qtab_hbm)

    return k(ptab, qtab, idx2)


def _sc_segment_sum(vals, idx2, zero_init, width):
    colw = width // NC
    per_s = EP // NS
    n_ch = per_s // CH
    n_it = n_ch // 2
    rows_s = N_NODES // NS

    @functools.partial(
        pl.kernel,
        out_type=jax.ShapeDtypeStruct((N_NODES, width), jnp.float32),
        mesh=_mesh(),
        compiler_params=pltpu.CompilerParams(use_tc_tiling_on_sc=False),
        scratch_types=[
            pltpu.VMEM((n_ch, CH), jnp.int32),
            pltpu.VMEM((2, CH, colw), jnp.float32),
            pltpu.VMEM_SHARED((N_NODES, colw), jnp.float32),
        ] + [pltpu.SemaphoreType.DMA] * 4,
    )
    def k(vals_hbm, dst_hbm, zero_hbm, out_hbm, idx_v, rows, acc, *sems):
        lsem = sems[:2]
        ssem = sems[2:]
        c = lax.axis_index("c")
        s = lax.axis_index("s")
        col0 = c * colw
        pltpu.sync_copy(dst_hbm.at[pl.ds(s * n_ch, n_ch)], idx_v)
        pltpu.sync_copy(zero_hbm.at[pl.ds(s * rows_s, rows_s)],
                        acc.at[pl.ds(s * rows_s, rows_s)])
        plsc.subcore_barrier()

        base = s * per_s

        def body(g, carry):
            for b in range(2):
                j = 2 * g + b

                @pl.when(g >= 1)
                def _():
                    pltpu.make_async_copy(
                        rows.at[b], acc.at[idx_v.at[j - 2]], ssem[b]).wait()

                pltpu.async_copy(
                    vals_hbm.at[pl.ds(base + j * CH, CH), pl.ds(col0, colw)],
                    rows.at[b], lsem[b])
            for b in range(2):
                j = 2 * g + b
                pltpu.make_async_copy(
                    vals_hbm.at[pl.ds(base + j * CH, CH), pl.ds(col0, colw)],
                    rows.at[b], lsem[b]).wait()
                pltpu.async_copy(rows.at[b], acc.at[idx_v.at[j]], ssem[b],
                                 add=True)
            return carry

        lax.fori_loop(0, n_it, body, 0, unroll=False)
        for b in range(2):
            j = (n_it - 1) * 2 + b
            pltpu.make_async_copy(rows.at[b], acc.at[idx_v.at[j]],
                                  ssem[b]).wait()
        plsc.subcore_barrier()
        pltpu.sync_copy(acc.at[pl.ds(s * rows_s, rows_s)],
                        out_hbm.at[pl.ds(s * rows_s, rows_s),
                                   pl.ds(col0, colw)])

    return k(vals, idx2, zero_init)


def _front_body(q_ref, r_ref, c_ref, de_ref, re_ref, ce_ref,
                w1_ref, b1_ref, w2_ref, b2_ref, w3_ref, b3_ref,
                w4_ref, b4_ref, mw1_ref, x_ref, po_ref, qo_ref):
    def onehot(iref):
        v = iref[...]
        return (v == lax.broadcasted_iota(jnp.int32, (BN, 16), 1)).astype(jnp.float32)

    e1 = de_ref[...] @ w1_ref[0:EMBED]
    e2 = re_ref[...] @ w1_ref[EMBED:2 * EMBED]
    e3 = ce_ref[...] @ w1_ref[2 * EMBED:3 * EMBED]
    z = onehot(q_ref) @ e1 + onehot(r_ref) @ e2 + onehot(c_ref) @ e3 + b1_ref[...]
    z = jax.nn.relu(z)
    z = jax.nn.relu(z @ w2_ref[...] + b2_ref[...])
    z = jax.nn.relu(z @ w3_ref[...] + b3_ref[...])
    x = z @ w4_ref[...] + b4_ref[...]
    x_ref[...] = x
    po_ref[...] = (x @ mw1_ref[0:HID]).astype(jnp.bfloat16)
    qo_ref[...] = (x @ mw1_ref[HID:2 * HID]).astype(jnp.bfloat16)


def _front(q2, r2, c2, de, re, ce, w1, b1, w2, b2, w3, b3, w4, b4, mw1):
    full = lambda a: pl.BlockSpec(a.shape, lambda i: (0,) * a.ndim)
    return pl.pallas_call(
        _front_body,
        grid=(NB_N,),
        in_specs=[
            pl.BlockSpec((BN, 1), lambda i: (i, 0)),
            pl.BlockSpec((BN, 1), lambda i: (i, 0)),
            pl.BlockSpec((BN, 1), lambda i: (i, 0)),
            full(de), full(re), full(ce),
            full(w1), full(b1), full(w2), full(b2), full(w3), full(b3),
            full(w4), full(b4), full(mw1),
        ],
        out_specs=[
            pl.BlockSpec((BN, HID), lambda i: (i, 0)),
            pl.BlockSpec((BN, HID), lambda i: (i, 0)),
            pl.BlockSpec((BN, HID), lambda i: (i, 0)),
        ],
        out_shape=[
            jax.ShapeDtypeStruct((N_NODES, HID), jnp.float32),
            jax.ShapeDtypeStruct((N_NODES, HID), jnp.bfloat16),
            jax.ShapeDtypeStruct((N_NODES, HID), jnp.bfloat16),
        ],
    )(q2, r2, c2, de, re, ce, w1, b1, w2, b2, w3, b3, w4, b4, mw1)


def _edge_body(gs_ref, gd_ref, b1_ref, w2_ref, b2_ref, w3_ref, b3_ref, out_ref):
    i = pl.program_id(0)
    z = jax.nn.relu(gs_ref[...].astype(jnp.float32)
                    + gd_ref[...].astype(jnp.float32) + b1_ref[...])
    z = jax.nn.relu(z @ w2_ref[...] + b2_ref[...])
    z = jax.nn.relu(z @ w3_ref[...] + b3_ref[...])
    out_ref[...] = jnp.where(i < NB_REAL, z, 0.0)


def _edge_mlp(g, b1, w2, b2, w3, b3):
    full = lambda a: pl.BlockSpec(a.shape, lambda i: (0,) * a.ndim)
    return pl.pallas_call(
        _edge_body,
        grid=(NB_E,),
        in_specs=[
            pl.BlockSpec((BE, HID), lambda i: (i, 0)),
            pl.BlockSpec((BE, HID), lambda i: (NB_E + i, 0)),
            full(b1), full(w2), full(b2), full(w3), full(b3),
        ],
        out_specs=pl.BlockSpec((BE, HID), lambda i: (i, 0)),
        out_shape=jax.ShapeDtypeStruct((EP, HID), jnp.float32),
    )(g, g, b1, w2, b2, w3, b3)


def _node_body(s_ref, cr_ref, x_ref, h_ref, c_ref,
               w4_ref, b4_ref, wih_ref, whh_ref, outw_ref, outb_ref, mw1_ref,
               nh_ref, nc_ref, p_ref, q_ref, lg_ref):
    S = s_ref[...]
    cnt = cr_ref[:, 0:1]
    m = S @ w4_ref[...] + cnt * b4_ref[...]
    x = x_ref[...]
    g = x @ wih_ref[0:HID] + m @ wih_ref[HID:2 * HID] + h_ref[...] @ whh_ref[...]
    i_g = jax.nn.sigmoid(g[:, 0:HID])
    f_g = jax.nn.sigmoid(g[:, HID:2 * HID])
    g_g = jnp.tanh(g[:, 2 * HID:3 * HID])
    o_g = jax.nn.sigmoid(g[:, 3 * HID:4 * HID])
    c2 = f_g * c_ref[...] + i_g * g_g
    h2 = o_g * jnp.tanh(c2)
    nh_ref[...] = h2
    nc_ref[...] = c2
    p_ref[...] = (h2 @ mw1_ref[0:HID]).astype(jnp.bfloat16)
    q_ref[...] = (h2 @ mw1_ref[HID:2 * HID]).astype(jnp.bfloat16)
    lg_ref[...] = h2 @ outw_ref[...] + outb_ref[...]


def _node_update(s, cr, x, h, c, w4, b4, wih, whh, outw, outb, mw1):
    full = lambda a: pl.BlockSpec(a.shape, lambda i: (0,) * a.ndim)
    return pl.pallas_call(
        _node_body,
        grid=(NB_N,),
        in_specs=[
            pl.BlockSpec((BN, HID), lambda i: (i, 0)),
            pl.BlockSpec((BN, CW), lambda i: (i, 0)),
            pl.BlockSpec((BN, HID), lambda i: (i, 0)),
            pl.BlockSpec((BN, HID), lambda i: (i, 0)),
            pl.BlockSpec((BN, HID), lambda i: (i, 0)),
            full(w4), full(b4), full(wih), full(whh),
            full(outw), full(outb), full(mw1),
        ],
        out_specs=[
            pl.BlockSpec((BN, HID), lambda i: (i, 0)),
            pl.BlockSpec((BN, HID), lambda i: (i, 0)),
            pl.BlockSpec((BN, HID), lambda i: (i, 0)),
            pl.BlockSpec((BN, HID), lambda i: (i, 0)),
            pl.BlockSpec((BN, 10), lambda i: (i, 0)),
        ],
        out_shape=[
            jax.ShapeDtypeStruct((N_NODES, HID), jnp.float32),
            jax.ShapeDtypeStruct((N_NODES, HID), jnp.float32),
            jax.ShapeDtypeStruct((N_NODES, HID), jnp.bfloat16),
            jax.ShapeDtypeStruct((N_NODES, HID), jnp.bfloat16),
            jax.ShapeDtypeStruct((N_NODES, 10), jnp.float32),
        ],
    )(s, cr, x, h, c, w4, b4, wih, whh, outw, outb, mw1)


def kernel(q, row, col, edge_index, params):
    p = params
    f32 = jnp.float32
    i32 = jnp.int32

    src = edge_index[0].astype(i32)
    dst = edge_index[1].astype(i32)
    pad = EP - N_EDGES
    zpad = jnp.zeros((pad,), i32)
    srcp = jnp.concatenate([src, zpad])
    dstp = jnp.concatenate([dst, zpad])
    gidx = jnp.concatenate([srcp, dstp])

    q2 = q.astype(i32).reshape(N_NODES, 1)
    r2 = row.astype(i32).reshape(N_NODES, 1)
    c2 = col.astype(i32).reshape(N_NODES, 1)
    de = jnp.zeros((16, EMBED), f32).at[0:10].set(p['digit_embed'])
    re = jnp.zeros((16, EMBED), f32).at[0:9].set(p['row_embed'])
    ce = jnp.zeros((16, EMBED), f32).at[0:9].set(p['col_embed'])
    rowb = lambda b: b.reshape(1, -1)

    zeros_h = jnp.zeros((N_NODES, HID), f32)
    zeros_h2 = jnp.zeros((N_NODES, HID // NC), f32)
    zeros_c2 = jnp.zeros((N_NODES, CW // NC), f32)
    ones_c = jnp.concatenate([jnp.ones((N_EDGES, CW), f32),
                              jnp.zeros((pad, CW), f32)])
    gidx2 = gidx.reshape(2 * EP // CH, CH)
    dst2 = dstp.reshape(EP // CH, CH)

    cntraw = _sc_segment_sum(ones_c, dst2, zeros_c2, CW)

    x, ptab, qtab = _front(q2, r2, c2, de, re, ce,
                   p['in_W1'], rowb(p['in_b1']), p['in_W2'], rowb(p['in_b2']),
                   p['in_W3'], rowb(p['in_b3']), p['in_W4'], rowb(p['in_b4']),
                   p['msg_W1'])

    h = zeros_h
    c = zeros_h
    logits_steps = []
    for _ in range(NUM_STEPS):
        g = _sc_gather(ptab, qtab, gidx2, 2 * EP)
        z3 = _edge_mlp(g, rowb(p['msg_b1']), p['msg_W2'], rowb(p['msg_b2']),
                       p['msg_W3'], rowb(p['msg_b3']))
        s = _sc_segment_sum(z3, dst2, zeros_h2, HID)
        h, c, ptab, qtab, lg = _node_update(
            s, cntraw, x, h, c, p['msg_W4'], rowb(p['msg_b4']),
            p['lstm_Wih'], p['lstm_Whh'], p['out_W'], rowb(p['out_b']),
            p['msg_W1'])
        logits_steps.append(lg)

    return jnp.stack(logits_steps, axis=0)


# speedup vs baseline: 1.3363x; 1.3363x over previous
import functools

import jax
import jax.numpy as jnp
from jax import lax
from jax.experimental import pallas as pl
from jax.experimental.pallas import tpu as pltpu
from jax.experimental.pallas import tpu_sc as plsc

NUM_STEPS = 8
EMBED = 16
HID = 96
HW = 128
N_NODES = 20736
N_EDGES = 414720

NC = 2
NS = 16
NW = NC * NS
CH = 128

BE = 1024
EP = 417792
NB_E = EP // BE
NB_REAL = N_EDGES // BE
BN = 256
NB_N = N_NODES // BN

NHALF = N_NODES // 2
ACC_ROWS = NHALF + 128
GUF = 4

_SC_PARAMS = pltpu.CompilerParams(use_tc_tiling_on_sc=False)


def _mesh():
    return plsc.VectorSubcoreMesh(core_axis_name="c", subcore_axis_name="s",
                                  num_cores=NC, num_subcores=NS)


def _sc_gather(ptab, qtab, idx2, n_rows):
    per_w = n_rows // NW
    n_ch = per_w // CH
    n_it = n_ch // GUF
    dt = ptab.dtype

    @functools.partial(
        pl.kernel,
        out_type=jax.ShapeDtypeStruct((n_rows, HW), dt),
        mesh=_mesh(),
        compiler_params=_SC_PARAMS,
        scratch_types=[
            pltpu.VMEM((n_ch, CH), jnp.int32),
            pltpu.VMEM((GUF, CH, HW), dt),
        ] + [pltpu.SemaphoreType.DMA] * (2 * GUF),
    )
    def k(ptab_hbm, qtab_hbm, idx_hbm, out_hbm, idx_v, rows, *sems):
        gsem = sems[:GUF]
        wsem = sems[GUF:]
        wid = lax.axis_index("s") * NC + lax.axis_index("c")
        base = wid * per_w
        pltpu.sync_copy(idx_hbm.at[pl.ds(wid * n_ch, n_ch)], idx_v)

        def run(table_hbm):
            def body(g, carry):
                for b in range(GUF):
                    j = g * GUF + b

                    @pl.when(g >= 1)
                    def _():
                        off = base + (j - GUF) * CH
                        pltpu.make_async_copy(
                            rows.at[b], out_hbm.at[pl.ds(off, CH)],
                            wsem[b]).wait()

                    pltpu.async_copy(table_hbm.at[idx_v.at[j]], rows.at[b],
                                     gsem[b])
                for b in range(GUF):
                    j = g * GUF + b
                    pltpu.make_async_copy(
                        table_hbm.at[idx_v.at[j]], rows.at[b], gsem[b]).wait()
                    pltpu.async_copy(
                        rows.at[b], out_hbm.at[pl.ds(base + j * CH, CH)],
                        wsem[b])
                return carry

            lax.fori_loop(0, n_it, body, 0, unroll=False)
            for b in range(GUF):
                j = (n_it - 1) * GUF + b
                pltpu.make_async_copy(
                    rows.at[b], out_hbm.at[pl.ds(base + j * CH, CH)],
                    wsem[b]).wait()

        @pl.when(wid < NW // 2)
        def _():
            run(ptab_hbm)

        @pl.when(wid >= NW // 2)
        def _():
            run(qtab_hbm)

    return k(ptab, qtab, idx2)


def _sc_segment_sum(vals, idxT, zero_init):
    per_s = EP // NS
    n_ch = per_s // CH
    n_it = n_ch // 2
    zrows = ACC_ROWS // NS
    orows = NHALF // NS

    @functools.partial(
        pl.kernel,
        out_type=jax.ShapeDtypeStruct((N_NODES, HW), jnp.float32),
        mesh=_mesh(),
        compiler_params=_SC_PARAMS,
        scratch_types=[
            pltpu.VMEM((2, CH), jnp.int32),
            pltpu.VMEM((2, CH, HW), jnp.float32),
            pltpu.VMEM_SHARED((ACC_ROWS, HW), jnp.float32),
        ] + [pltpu.SemaphoreType.DMA] * 6,
    )
    def k(vals_hbm, idx_hbm, zero_hbm, out_hbm, idx_v, rows, acc, *sems):
        isem = sems[0:2]
        lsem = sems[2:4]
        ssem = sems[4:6]
        c = lax.axis_index("c")
        s = lax.axis_index("s")
        pltpu.sync_copy(zero_hbm.at[pl.ds(s * zrows, zrows)],
                        acc.at[pl.ds(s * zrows, zrows)])
        plsc.subcore_barrier()

        base = s * per_s
        cbase = s * n_ch

        def body(g, carry):
            for b in range(2):
                j = 2 * g + b

                @pl.when(g >= 1)
                def _():
                    pltpu.make_async_copy(
                        rows.at[b], acc.at[idx_v.at[b]], ssem[b]).wait()

                pltpu.async_copy(idx_hbm.at[c, cbase + j], idx_v.at[b],
                                 isem[b])
                pltpu.async_copy(vals_hbm.at[pl.ds(base + j * CH, CH)],
                                 rows.at[b], lsem[b])
            for b in range(2):
                j = 2 * g + b
                pltpu.make_async_copy(idx_hbm.at[c, cbase + j],
                                      idx_v.at[b], isem[b]).wait()
                pltpu.make_async_copy(vals_hbm.at[pl.ds(base + j * CH, CH)],
                                      rows.at[b], lsem[b]).wait()
                pltpu.async_copy(rows.at[b], acc.at[idx_v.at[b]], ssem[b],
                                 add=True)
            return carry

        lax.fori_loop(0, n_it, body, 0, unroll=False)
        for b in range(2):
            pltpu.make_async_copy(rows.at[b], acc.at[idx_v.at[b]],
                                  ssem[b]).wait()
        plsc.subcore_barrier()
        pltpu.sync_copy(acc.at[pl.ds(s * orows, orows)],
                        out_hbm.at[pl.ds(c * NHALF + s * orows, orows)])

    return k(vals, idxT, zero_init)


def _front_body(q_ref, r_ref, c_ref, de_ref, re_ref, ce_ref,
                w1_ref, b1_ref, w2_ref, b2_ref, w3_ref, b3_ref,
                w4_ref, b4_ref, mw1_ref, x_ref, po_ref, qo_ref):
    def onehot(iref):
        v = iref[...]
        return (v == lax.broadcasted_iota(jnp.int32, (BN, 16), 1)).astype(jnp.float32)

    e1 = de_ref[...] @ w1_ref[0:EMBED]
    e2 = re_ref[...] @ w1_ref[EMBED:2 * EMBED]
    e3 = ce_ref[...] @ w1_ref[2 * EMBED:3 * EMBED]
    z = onehot(q_ref) @ e1 + onehot(r_ref) @ e2 + onehot(c_ref) @ e3 + b1_ref[...]
    z = jax.nn.relu(z)
    z = jax.nn.relu(z @ w2_ref[...] + b2_ref[...])
    z = jax.nn.relu(z @ w3_ref[...] + b3_ref[...])
    x = z @ w4_ref[...] + b4_ref[...]
    x_ref[...] = x
    po_ref[...] = x @ mw1_ref[0:HID]
    qo_ref[...] = x @ mw1_ref[HID:2 * HID]


def _front(q2, r2, c2, de, re, ce, w1, b1, w2, b2, w3, b3, w4, b4, mw1p):
    full = lambda a: pl.BlockSpec(a.shape, lambda i: (0,) * a.ndim)
    return pl.pallas_call(
        _front_body,
        grid=(NB_N,),
        in_specs=[
            pl.BlockSpec((BN, 1), lambda i: (i, 0)),
            pl.BlockSpec((BN, 1), lambda i: (i, 0)),
            pl.BlockSpec((BN, 1), lambda i: (i, 0)),
            full(de), full(re), full(ce),
            full(w1), full(b1), full(w2), full(b2), full(w3), full(b3),
            full(w4), full(b4), full(mw1p),
        ],
        out_specs=[
            pl.BlockSpec((BN, HID), lambda i: (i, 0)),
            pl.BlockSpec((BN, HW), lambda i: (i, 0)),
            pl.BlockSpec((BN, HW), lambda i: (i, 0)),
        ],
        out_shape=[
            jax.ShapeDtypeStruct((N_NODES, HID), jnp.float32),
            jax.ShapeDtypeStruct((N_NODES, HW), jnp.float32),
            jax.ShapeDtypeStruct((N_NODES, HW), jnp.float32),
        ],
    )(q2, r2, c2, de, re, ce, w1, b1, w2, b2, w3, b3, w4, b4, mw1p)


def _edge_body(gs_ref, gd_ref, b1_ref, w2_ref, b2_ref, w3_ref, b3_ref, out_ref):
    i = pl.program_id(0)
    z = jax.nn.relu(gs_ref[...] + gd_ref[...] + b1_ref[...])
    z = jax.nn.relu(z @ w2_ref[...] + b2_ref[...])
    z = jax.nn.relu(z @ w3_ref[...] + b3_ref[...])
    out_ref[...] = jnp.where(i < NB_REAL, z, 0.0)


def _edge_mlp(g, b1, w2, b2, w3, b3):
    full = lambda a: pl.BlockSpec(a.shape, lambda i: (0,) * a.ndim)
    return pl.pallas_call(
        _edge_body,
        grid=(NB_E,),
        in_specs=[
            pl.BlockSpec((BE, HW), lambda i: (i, 0)),
            pl.BlockSpec((BE, HW), lambda i: (NB_E + i, 0)),
            full(b1), full(w2), full(b2), full(w3), full(b3),
        ],
        out_specs=pl.BlockSpec((BE, HW), lambda i: (i, 0)),
        out_shape=jax.ShapeDtypeStruct((EP, HW), jnp.float32),
    )(g, g, b1, w2, b2, w3, b3)


def _node_body(s_ref, cr_ref, x_ref, h_ref, c_ref,
               w4_ref, b4_ref, wih_ref, whh_ref, outw_ref, outb_ref, mw1_ref,
               nh_ref, nc_ref, p_ref, q_ref, lg_ref):
    S = s_ref[...]
    cnt = cr_ref[:, 0:1]
    m = S @ w4_ref[...] + cnt * b4_ref[...]
    x = x_ref[...]
    g = x @ wih_ref[0:HID] + m @ wih_ref[HID:2 * HID] + h_ref[...] @ whh_ref[...]
    i_g = jax.nn.sigmoid(g[:, 0:HID])
    f_g = jax.nn.sigmoid(g[:, HID:2 * HID])
    g_g = jnp.tanh(g[:, 2 * HID:3 * HID])
    o_g = jax.nn.sigmoid(g[:, 3 * HID:4 * HID])
    c2 = f_g * c_ref[...] + i_g * g_g
    h2 = o_g * jnp.tanh(c2)
    nh_ref[...] = h2
    nc_ref[...] = c2
    p_ref[...] = h2 @ mw1_ref[0:HID]
    q_ref[...] = h2 @ mw1_ref[HID:2 * HID]
    lg_ref[...] = h2 @ outw_ref[...] + outb_ref[...]


def _node_update(s, cr, x, h, c, w4p, b4, wih, whh, outw, outb, mw1p):
    full = lambda a: pl.BlockSpec(a.shape, lambda i: (0,) * a.ndim)
    return pl.pallas_call(
        _node_body,
        grid=(NB_N,),
        in_specs=[
            pl.BlockSpec((BN, HW), lambda i: (i, 0)),
            pl.BlockSpec((BN, HW), lambda i: (i, 0)),
            pl.BlockSpec((BN, HID), lambda i: (i, 0)),
            pl.BlockSpec((BN, HID), lambda i: (i, 0)),
            pl.BlockSpec((BN, HID), lambda i: (i, 0)),
            full(w4p), full(b4), full(wih), full(whh),
            full(outw), full(outb), full(mw1p),
        ],
        out_specs=[
            pl.BlockSpec((BN, HID), lambda i: (i, 0)),
            pl.BlockSpec((BN, HID), lambda i: (i, 0)),
            pl.BlockSpec((BN, HW), lambda i: (i, 0)),
            pl.BlockSpec((BN, HW), lambda i: (i, 0)),
            pl.BlockSpec((BN, 10), lambda i: (i, 0)),
        ],
        out_shape=[
            jax.ShapeDtypeStruct((N_NODES, HID), jnp.float32),
            jax.ShapeDtypeStruct((N_NODES, HID), jnp.float32),
            jax.ShapeDtypeStruct((N_NODES, HW), jnp.float32),
            jax.ShapeDtypeStruct((N_NODES, HW), jnp.float32),
            jax.ShapeDtypeStruct((N_NODES, 10), jnp.float32),
        ],
    )(s, cr, x, h, c, w4p, b4, wih, whh, outw, outb, mw1p)


def kernel(q, row, col, edge_index, params):
    p = params
    f32 = jnp.float32
    i32 = jnp.int32

    src = edge_index[0].astype(i32)
    dst = edge_index[1].astype(i32)
    pad = EP - N_EDGES
    zpad = jnp.zeros((pad,), i32)
    srcp = jnp.concatenate([src, zpad])
    dstp = jnp.concatenate([dst, zpad])
    gidx2 = jnp.concatenate([srcp, dstp]).reshape(2 * EP // CH, CH)
    d0 = jnp.where(dstp < NHALF, dstp, NHALF)
    d1 = jnp.where(dstp >= NHALF, dstp - NHALF, NHALF)
    idxT = jnp.stack([d0, d1]).reshape(NC, EP // CH, CH)

    q2 = q.astype(i32).reshape(N_NODES, 1)
    r2 = row.astype(i32).reshape(N_NODES, 1)
    c2 = col.astype(i32).reshape(N_NODES, 1)
    de = jnp.zeros((16, EMBED), f32).at[0:10].set(p['digit_embed'])
    re = jnp.zeros((16, EMBED), f32).at[0:9].set(p['row_embed'])
    ce = jnp.zeros((16, EMBED), f32).at[0:9].set(p['col_embed'])
    rowb = lambda b: b.reshape(1, -1)
    padw = lambda w, r, cdim: jnp.zeros((r, cdim), f32).at[
        0:w.shape[0], 0:w.shape[1]].set(w)

    mw1p = padw(p['msg_W1'], 2 * HID, HW)
    w2p = padw(p['msg_W2'], HW, HW)
    w3p = padw(p['msg_W3'], HW, HW)
    w4p = padw(p['msg_W4'], HW, HID)
    b1p = padw(rowb(p['msg_b1']), 1, HW)
    b2p = padw(rowb(p['msg_b2']), 1, HW)
    b3p = padw(rowb(p['msg_b3']), 1, HW)

    zeros_h = jnp.zeros((N_NODES, HID), f32)
    zeros_acc = jnp.zeros((ACC_ROWS, HW), f32)
    ones_e = jnp.concatenate([jnp.ones((N_EDGES, HW), f32),
                              jnp.zeros((pad, HW), f32)])

    cntraw = _sc_segment_sum(ones_e, idxT, zeros_acc)

    x, ptab, qtab = _front(q2, r2, c2, de, re, ce,
                           p['in_W1'], rowb(p['in_b1']),
                           p['in_W2'], rowb(p['in_b2']),
                           p['in_W3'], rowb(p['in_b3']),
                           p['in_W4'], rowb(p['in_b4']), mw1p)

    h = zeros_h
    c = zeros_h
    logits_steps = []
    for _ in range(NUM_STEPS):
        g = _sc_gather(ptab, qtab, gidx2, 2 * EP)
        z3 = _edge_mlp(g, b1p, w2p, b2p, w3p, b3p)
        s = _sc_segment_sum(z3, idxT, zeros_acc)
        h, c, ptab, qtab, lg = _node_update(
            s, cntraw, x, h, c, w4p, rowb(p['msg_b4']),
            p['lstm_Wih'], p['lstm_Whh'], p['out_W'], rowb(p['out_b']), mw1p)
        logits_steps.append(lg)

    return jnp.stack(logits_steps, axis=0)
